# baseline (speedup 1.0000x reference)
import functools

import jax
import jax.numpy as jnp
from jax import lax
from jax.experimental import pallas as pl
from jax.experimental.pallas import tpu as pltpu
from jax.experimental.pallas import tpu_sc as plsc

NUM_USERS = 1000000
EMBED_DIM = 64
BATCH = 16384
_BLK = 128
_BLK_SHIFT = 7
_NB = (NUM_USERS + _BLK - 1) // _BLK
_SENT = _NB + 7

_info = plsc.get_sparse_core_info()
_NC, _NS, _L = _info.num_cores, _info.num_subcores, _info.num_lanes
_NW = _NC * _NS
_Q, _R = divmod(_NB, _NW)
_GROUPS = EMBED_DIM // _L
_NBUF = 4
_RING = 64

_mesh = plsc.VectorSubcoreMesh(core_axis_name="c", subcore_axis_name="s")


@functools.partial(
    pl.kernel,
    mesh=_mesh,
    out_type=jax.ShapeDtypeStruct((BATCH, EMBED_DIM), jnp.float32),
    scratch_types=[
        pltpu.VMEM((BATCH,), jnp.int32),
        pltpu.VMEM((BATCH + _L,), jnp.int32),
        pltpu.VMEM((BATCH + _L,), jnp.int32),
        pltpu.VMEM((_RING, EMBED_DIM), jnp.float32),
        pltpu.VMEM((EMBED_DIM,), jnp.float32),
    ]
    + [pltpu.VMEM((EMBED_DIM, _BLK), jnp.float32) for _ in range(_NBUF)]
    + [pltpu.SemaphoreType.DMA for _ in range(_NBUF)]
    + [pltpu.SemaphoreType.DMA],
    compiler_params=pltpu.CompilerParams(needs_layout_passes=False),
)
def _sc_lookup(table_t_hbm, idx_hbm, rel_hbm, out_hbm,
               idx_v, uc, pc, ring, rel_v,
               b0, b1, b2, b3, s0, s1, s2, s3, rsem):
    bufs = (b0, b1, b2, b3)
    sems = (s0, s1, s2, s3)
    wid = lax.axis_index("s") * _NC + lax.axis_index("c")

    minw = lax.min(wid, _R)
    S = wid * _Q + minw
    bc = _Q + jnp.where(wid < _R, 1, 0)
    lo = S * _BLK
    hi = lax.min((S + bc) * _BLK, NUM_USERS)

    c0 = lax.iota(jnp.int32, _L)
    ones = c0 * 0 + 1

    def fire_block(slot, blk):
        blk = lax.min(blk, S + bc - 1)
        bstart = pl.multiple_of(blk * _BLK, _BLK)
        pltpu.make_async_copy(
            table_t_hbm.at[:, pl.ds(bstart, _BLK)], bufs[slot], sems[slot]
        ).start()

    for k in range(_NBUF):
        fire_block(k, S + k)

    pltpu.sync_copy(idx_hbm, idx_v)
    pltpu.sync_copy(rel_hbm, rel_v)
    rel_regs = [rel_v[pl.ds(g * _L, _L)] for g in range(_GROUPS)]

    pad = ones * (_SENT * _BLK)

    def fill_sent(v, carry):
        uc[pl.ds(v * _L, _L)] = pad
        return carry

    lax.fori_loop(0, (BATCH + _L) // _L, fill_sent, 0)

    def scan_idx(v, off):
        u = idx_v[pl.ds(v * _L, _L)]
        m = (u >= lo) & (u < hi)
        plsc.store_compressed(uc.at[pl.ds(off, _L)], u, mask=m)
        plsc.store_compressed(pc.at[pl.ds(off, _L)], ones * (v * _L) + c0,
                              mask=m)
        return off + plsc.all_reduce_population_count(m)[0]

    n_w = lax.fori_loop(0, BATCH // _L, scan_idx, 0)
    plsc.store_compressed(uc.at[pl.ds(n_w, _L)], pad, mask=ones > 0)
    nv = (n_w + _L - 1) // _L

    def row_drain(_, w):
        pltpu.make_async_copy(
            ring.at[pl.ds(0, 1)], out_hbm.at[pl.ds(0, 1)], rsem
        ).wait()
        return w + 1

    def chunk(c, carry):
        counter, waited = carry
        for k in range(_NBUF):
            blk = c * _NBUF + k
            pltpu.make_async_copy(
                table_t_hbm.at[:, pl.ds(0, _BLK)], bufs[k], sems[k]
            ).wait()
            live = (blk < bc).astype(jnp.int32)
            blk_id = S + blk

            def match(t, carry2, k=k, blk_id=blk_id, live=live):
                counter2, waited2 = carry2
                u = uc[pl.ds(t * _L, _L)]
                p = pc[pl.ds(t * _L, _L)]
                m = lax.shift_right_logical(u, _BLK_SHIFT) == (ones * blk_id)
                m = m & (ones * live > 0)
                cnt = plsc.all_reduce_population_count(m)[0]

                def extract(cc):
                    counter3, waited3 = cc
                    target = lax.max(waited3, counter3 + cnt - _RING)
                    waited3 = lax.fori_loop(0, target - waited3, row_drain,
                                            waited3)
                    slotv = lax.rem(
                        counter3 + plsc.cumsum(m.astype(jnp.int32)) - 1, _RING)
                    mi = m.astype(jnp.int32)
                    for j in range(_L):
                        @pl.when(mi[j] == 1)
                        def _():
                            slot = slotv[j]
                            colv = ones * lax.rem(u[j], _BLK)
                            for g in range(_GROUPS):
                                vals = plsc.load_gather(
                                    bufs[k], [c0 + g * _L, colv])
                                ring[slot, pl.ds(g * _L, _L)] = (
                                    vals + rel_regs[g])
                            pltpu.make_async_copy(
                                ring.at[pl.ds(slot, 1)],
                                out_hbm.at[pl.ds(p[j], 1)],
                                rsem,
                            ).start()
                    return counter3 + cnt, waited3

                return extract((counter2, waited2))

            counter, waited = lax.fori_loop(0, nv, match, (counter, waited))
            fire_block(k, S + blk + _NBUF)
        return counter, waited

    nchunks = (bc + _NBUF - 1) // _NBUF
    counter, waited = lax.fori_loop(0, nchunks, chunk, (0, 0))

    lax.fori_loop(0, counter - waited, row_drain, waited)
    for k in range(_NBUF):
        pltpu.make_async_copy(
            table_t_hbm.at[:, pl.ds(0, _BLK)], bufs[k], sems[k]
        ).wait()


def kernel(batch_users, user_emb_weight, rel_emb_weight):
    idx = batch_users.astype(jnp.int32)
    table_t = jnp.swapaxes(user_emb_weight, 0, 1)
    rel_row = rel_emb_weight[-1]
    return _sc_lookup(table_t, idx, rel_row)


# speedup vs baseline: 5.9454x; 5.9454x over previous
import functools

import jax
import jax.numpy as jnp
from jax import lax
from jax.experimental import pallas as pl
from jax.experimental.pallas import tpu as pltpu
from jax.experimental.pallas import tpu_sc as plsc

NUM_USERS = 1000000
EMBED_DIM = 64
BATCH = 16384
_BLK = 128
_BLK_SHIFT = 7
_NB = (NUM_USERS + _BLK - 1) // _BLK
_SENT = _NB + 7

_info = plsc.get_sparse_core_info()
_NC, _NS, _L = _info.num_cores, _info.num_subcores, _info.num_lanes
_NW = _NC * _NS
_Q, _R = divmod(_NB, _NW)
_GROUPS = EMBED_DIM // _L
_NBUF = 4
_RING = 64

_mesh = plsc.VectorSubcoreMesh(core_axis_name="c", subcore_axis_name="s")


@functools.partial(
    pl.kernel,
    mesh=_mesh,
    out_type=jax.ShapeDtypeStruct((BATCH, EMBED_DIM), jnp.float32),
    scratch_types=[
        pltpu.VMEM((BATCH,), jnp.int32),
        pltpu.VMEM((BATCH + _L,), jnp.int32),
        pltpu.VMEM((BATCH + _L,), jnp.int32),
        pltpu.VMEM((_RING, EMBED_DIM), jnp.float32),
        pltpu.VMEM((EMBED_DIM,), jnp.float32),
    ]
    + [pltpu.VMEM((EMBED_DIM, _BLK), jnp.float32) for _ in range(_NBUF)]
    + [pltpu.SemaphoreType.DMA for _ in range(_NBUF)]
    + [pltpu.SemaphoreType.DMA],
    compiler_params=pltpu.CompilerParams(needs_layout_passes=False),
)
def _sc_lookup(table_t_hbm, idx_hbm, rel_hbm, out_hbm,
               idx_v, uc, pc, ring, rel_v,
               b0, b1, b2, b3, s0, s1, s2, s3, rsem):
    bufs = (b0, b1, b2, b3)
    sems = (s0, s1, s2, s3)
    wid = lax.axis_index("s") * _NC + lax.axis_index("c")

    minw = lax.min(wid, _R)
    S = wid * _Q + minw
    bc = _Q + jnp.where(wid < _R, 1, 0)
    lo = S * _BLK
    hi = lax.min((S + bc) * _BLK, NUM_USERS)

    c0 = lax.iota(jnp.int32, _L)
    ones = c0 * 0 + 1

    def fire_block(slot, blk):
        blk = lax.min(blk, S + bc - 1)
        bstart = pl.multiple_of(blk * _BLK, _BLK)
        pltpu.make_async_copy(
            table_t_hbm.at[:, pl.ds(bstart, _BLK)], bufs[slot], sems[slot]
        ).start()

    for k in range(_NBUF):
        fire_block(k, S + k)

    pltpu.sync_copy(idx_hbm, idx_v)
    pltpu.sync_copy(rel_hbm, rel_v)
    rel_regs = [rel_v[pl.ds(g * _L, _L)] for g in range(_GROUPS)]

    pad = ones * (_SENT * _BLK)

    def fill_sent(v, carry):
        uc[pl.ds(v * _L, _L)] = pad
        return carry

    lax.fori_loop(0, (BATCH + _L) // _L, fill_sent, 0)

    def scan_idx(v, off):
        u = idx_v[pl.ds(v * _L, _L)]
        m = (u >= lo) & (u < hi)
        plsc.store_compressed(uc.at[pl.ds(off, _L)], u, mask=m)
        plsc.store_compressed(pc.at[pl.ds(off, _L)], ones * (v * _L) + c0,
                              mask=m)
        return off + plsc.all_reduce_population_count(m)[0]

    n_w = lax.fori_loop(0, BATCH // _L, scan_idx, 0)
    plsc.store_compressed(uc.at[pl.ds(n_w, _L)], pad, mask=ones > 0)
    nv = (n_w + _L - 1) // _L

    def row_drain(_, w):
        pltpu.make_async_copy(
            ring.at[pl.ds(0, 1)], out_hbm.at[pl.ds(0, 1)], rsem
        ).wait()
        return w + 1

    def chunk(c, carry):
        counter, waited = carry
        for k in range(_NBUF):
            blk = c * _NBUF + k
            pltpu.make_async_copy(
                table_t_hbm.at[:, pl.ds(0, _BLK)], bufs[k], sems[k]
            ).wait()
            live = (blk < bc).astype(jnp.int32)
            blk_id = S + blk

            def match(t, carry2, k=k, blk_id=blk_id, live=live):
                counter2, waited2 = carry2
                u = uc[pl.ds(t * _L, _L)]
                p = pc[pl.ds(t * _L, _L)]
                m = lax.shift_right_logical(u, _BLK_SHIFT) == (ones * blk_id)
                m = m & (ones * live > 0)
                cnt = plsc.all_reduce_population_count(m)[0]

                def extract(cc):
                    counter3, waited3 = cc
                    target = lax.max(waited3, counter3 + cnt - _RING)
                    waited3 = lax.fori_loop(0, target - waited3, row_drain,
                                            waited3)
                    slotv = lax.rem(
                        counter3 + plsc.cumsum(m.astype(jnp.int32)) - 1, _RING)
                    mi = m.astype(jnp.int32)
                    for j in range(_L):
                        @pl.when(mi[j] == 1)
                        def _():
                            slot = slotv[j]
                            colv = ones * lax.rem(u[j], _BLK)
                            for g in range(_GROUPS):
                                vals = plsc.load_gather(
                                    bufs[k], [c0 + g * _L, colv])
                                ring[slot, pl.ds(g * _L, _L)] = (
                                    vals + rel_regs[g])
                            pltpu.make_async_copy(
                                ring.at[pl.ds(slot, 1)],
                                out_hbm.at[pl.ds(p[j], 1)],
                                rsem,
                            ).start()
                    return counter3 + cnt, waited3

                return lax.cond(cnt > 0, extract, lambda cc: cc,
                                (counter2, waited2))

            counter, waited = lax.fori_loop(0, nv, match, (counter, waited))
            fire_block(k, S + blk + _NBUF)
        return counter, waited

    nchunks = (bc + _NBUF - 1) // _NBUF
    counter, waited = lax.fori_loop(0, nchunks, chunk, (0, 0))

    lax.fori_loop(0, counter - waited, row_drain, waited)
    for k in range(_NBUF):
        pltpu.make_async_copy(
            table_t_hbm.at[:, pl.ds(0, _BLK)], bufs[k], sems[k]
        ).wait()


def kernel(batch_users, user_emb_weight, rel_emb_weight):
    idx = batch_users.astype(jnp.int32)
    table_t = jnp.swapaxes(user_emb_weight, 0, 1)
    rel_row = rel_emb_weight[-1]
    return _sc_lookup(table_t, idx, rel_row)


# speedup vs baseline: 8.8638x; 1.4909x over previous
import functools

import jax
import jax.numpy as jnp
from jax import lax
from jax.experimental import pallas as pl
from jax.experimental.pallas import tpu as pltpu
from jax.experimental.pallas import tpu_sc as plsc

NUM_USERS = 1000000
EMBED_DIM = 64
BATCH = 16384
_BLK = 128
_NB = (NUM_USERS + _BLK - 1) // _BLK

_info = plsc.get_sparse_core_info()
_NC, _NS, _L = _info.num_cores, _info.num_subcores, _info.num_lanes
_NW = _NC * _NS
_Q, _R = divmod(_NB, _NW)
_GROUPS = EMBED_DIM // _L
_NBUF = 4
_RING = 32
_NSUB = 16
_SUBSZ = (_Q + _NSUB) // _NSUB
_POSB = 14
_SENT = 0xFFFF << _POSB
_GCAP = BATCH + _NSUB * 2 * _L + _L

_mesh = plsc.VectorSubcoreMesh(core_axis_name="c", subcore_axis_name="s")


@functools.partial(
    pl.kernel,
    mesh=_mesh,
    out_type=jax.ShapeDtypeStruct((BATCH, EMBED_DIM), jnp.float32),
    scratch_types=[
        pltpu.VMEM((_GCAP,), jnp.int32),
        pltpu.VMEM((BATCH + _L,), jnp.int32),
        pltpu.VMEM((_RING, EMBED_DIM), jnp.float32),
        pltpu.VMEM((EMBED_DIM,), jnp.float32),
        pltpu.SMEM((_NSUB + 2,), jnp.int32),
    ]
    + [pltpu.VMEM((EMBED_DIM, _BLK), jnp.float32) for _ in range(_NBUF)]
    + [pltpu.SemaphoreType.DMA for _ in range(_NBUF)]
    + [pltpu.SemaphoreType.DMA],
    compiler_params=pltpu.CompilerParams(needs_layout_passes=False),
)
def _sc_lookup(table_t_hbm, idx_hbm, rel_hbm, out_hbm,
               gv, uc, ring, rel_v, seg_s,
               b0, b1, b2, b3, s0, s1, s2, s3, rsem):
    bufs = (b0, b1, b2, b3)
    sems = (s0, s1, s2, s3)
    wid = lax.axis_index("s") * _NC + lax.axis_index("c")

    minw = lax.min(wid, _R)
    S = wid * _Q + minw
    bc = _Q + jnp.where(wid < _R, 1, 0)
    lo = S * _BLK
    hi = lax.min((S + bc) * _BLK, NUM_USERS)

    c0 = lax.iota(jnp.int32, _L)
    ones = c0 * 0 + 1

    def fire_block(slot, blk):
        blk = lax.min(blk, bc - 1)
        bstart = pl.multiple_of((S + blk) * _BLK, _BLK)
        pltpu.make_async_copy(
            table_t_hbm.at[:, pl.ds(bstart, _BLK)], bufs[slot], sems[slot]
        ).start()

    for k in range(_NBUF):
        fire_block(k, k)

    pltpu.sync_copy(idx_hbm, gv.at[pl.ds(0, BATCH)])
    pltpu.sync_copy(rel_hbm, rel_v)
    rel_regs = [rel_v[pl.ds(g * _L, _L)] for g in range(_GROUPS)]

    def scan_idx(v, off):
        u = gv[pl.ds(v * _L, _L)]
        m = (u >= lo) & (u < hi)
        packed = lax.shift_left(u - lo, _POSB) | (ones * (v * _L) + c0)
        plsc.store_compressed(uc.at[pl.ds(off, _L)], packed, mask=m)
        return off + plsc.all_reduce_population_count(m)[0]

    n_w = lax.fori_loop(0, BATCH // _L, scan_idx, 0)
    plsc.store_compressed(uc.at[pl.ds(n_w, _L)], ones * _SENT, mask=ones > 0)
    nv = (n_w + _L - 1) // _L

    goff = 0
    for s in range(_NSUB):
        seg_s[s] = goff

        def group(t, off, s=s):
            pk = uc[pl.ds(t * _L, _L)]
            m = lax.shift_right_logical(pk, _POSB + 7 + 4) == (ones * s)
            plsc.store_compressed(gv.at[pl.ds(off, _L)], pk, mask=m)
            return off + plsc.all_reduce_population_count(m)[0]

        goff = lax.fori_loop(0, nv, group, goff)
        plsc.store_compressed(gv.at[pl.ds(goff, _L)], ones * _SENT,
                              mask=ones > 0)
        goff = ((goff >> 4) << 4) + _L
    seg_s[_NSUB] = goff

    def row_drain(_, w):
        pltpu.make_async_copy(
            ring.at[pl.ds(0, 1)], out_hbm.at[pl.ds(0, 1)], rsem
        ).wait()
        return w + 1

    def chunk(c, carry):
        counter, waited = carry
        for k in range(_NBUF):
            blk = c * _NBUF + k
            pltpu.make_async_copy(
                table_t_hbm.at[:, pl.ds(0, _BLK)], bufs[k], sems[k]
            ).wait()
            sub = blk // _SUBSZ
            seg_lo = seg_s[sub]
            nseg = (seg_s[sub + 1] - seg_lo) >> 4

            def match(t, carry2, k=k, blk=blk, seg_lo=seg_lo):
                counter2, waited2 = carry2
                pk = gv[pl.ds(seg_lo + t * _L, _L)]
                m = lax.shift_right_logical(pk, _POSB + 7) == (ones * blk)
                cnt = plsc.all_reduce_population_count(m)[0]

                def extract(cc):
                    counter3, waited3 = cc
                    target = lax.max(waited3, counter3 + cnt - _RING)
                    waited3 = lax.fori_loop(0, target - waited3, row_drain,
                                            waited3)
                    slotv = lax.rem(
                        counter3 + plsc.cumsum(m.astype(jnp.int32)) - 1, _RING)
                    mi = m.astype(jnp.int32)
                    for j in range(_L):
                        @pl.when(mi[j] == 1)
                        def _():
                            slot = slotv[j]
                            colv = ones * (
                                lax.shift_right_logical(pk[j], _POSB) & 127)
                            pos = pk[j] & ((1 << _POSB) - 1)
                            for g in range(_GROUPS):
                                vals = plsc.load_gather(
                                    bufs[k], [c0 + g * _L, colv])
                                ring[slot, pl.ds(g * _L, _L)] = (
                                    vals + rel_regs[g])
                            pltpu.make_async_copy(
                                ring.at[pl.ds(slot, 1)],
                                out_hbm.at[pl.ds(pos, 1)],
                                rsem,
                            ).start()
                    return counter3 + cnt, waited3

                return lax.cond(cnt > 0, extract, lambda cc: cc,
                                (counter2, waited2))

            counter, waited = lax.fori_loop(0, nseg, match, (counter, waited))
            fire_block(k, blk + _NBUF)
        return counter, waited

    nchunks = (bc + _NBUF - 1) // _NBUF
    counter, waited = lax.fori_loop(0, nchunks, chunk, (0, 0))

    lax.fori_loop(0, counter - waited, row_drain, waited)
    for k in range(_NBUF):
        pltpu.make_async_copy(
            table_t_hbm.at[:, pl.ds(0, _BLK)], bufs[k], sems[k]
        ).wait()


def kernel(batch_users, user_emb_weight, rel_emb_weight):
    idx = batch_users.astype(jnp.int32)
    table_t = jnp.swapaxes(user_emb_weight, 0, 1)
    rel_row = rel_emb_weight[-1]
    return _sc_lookup(table_t, idx, rel_row)


# speedup vs baseline: 14.5285x; 1.6391x over previous
import functools

import jax
import jax.numpy as jnp
from jax import lax
from jax.experimental import pallas as pl
from jax.experimental.pallas import tpu as pltpu
from jax.experimental.pallas import tpu_sc as plsc

NUM_USERS = 1000000
EMBED_DIM = 64
BATCH = 16384
_BLK = 128

_info = plsc.get_sparse_core_info()
_NC, _NS, _L = _info.num_cores, _info.num_subcores, _info.num_lanes
_NW = _NC * _NS
_BPW = BATCH // _NW
_NV = _BPW // _L
_GROUPS = EMBED_DIM // _L
_NBUF = 8

_mesh = plsc.VectorSubcoreMesh(core_axis_name="c", subcore_axis_name="s")


@functools.partial(
    pl.kernel,
    mesh=_mesh,
    out_type=jax.ShapeDtypeStruct((BATCH, EMBED_DIM), jnp.float32),
    scratch_types=[
        pltpu.VMEM((_BPW + _L,), jnp.int32),
        pltpu.VMEM((4 * _L, EMBED_DIM), jnp.float32),
        pltpu.VMEM((EMBED_DIM,), jnp.float32),
    ]
    + [pltpu.VMEM((EMBED_DIM, _BLK), jnp.float32) for _ in range(_NBUF)]
    + [pltpu.SemaphoreType.DMA for _ in range(_NBUF)]
    + [pltpu.SemaphoreType.DMA],
    compiler_params=pltpu.CompilerParams(needs_layout_passes=False),
)
def _sc_lookup(table_t_hbm, idx_hbm, rel_hbm, out_hbm,
               idx_v, rows_v, rel_v,
               b0, b1, b2, b3, b4, b5, b6, b7,
               s0, s1, s2, s3, s4, s5, s6, s7, sem):
    bufs = (b0, b1, b2, b3, b4, b5, b6, b7)
    sems = (s0, s1, s2, s3, s4, s5, s6, s7)
    wid = lax.axis_index("s") * _NC + lax.axis_index("c")
    base = wid * _BPW

    pltpu.sync_copy(idx_hbm.at[pl.ds(base, _BPW)], idx_v.at[pl.ds(0, _BPW)])
    pltpu.sync_copy(idx_hbm.at[pl.ds(0, _L)], idx_v.at[pl.ds(_BPW, _L)])
    pltpu.sync_copy(rel_hbm, rel_v)
    rel_regs = [rel_v[pl.ds(g * _L, _L)] for g in range(_GROUPS)]
    c0 = lax.iota(jnp.int32, _L)
    ones = c0 * 0 + 1

    def fire(slot, u):
        u = lax.min(lax.max(u, 0), NUM_USERS - 1)
        bstart = pl.multiple_of((u // _BLK) * _BLK, _BLK)
        pltpu.make_async_copy(
            table_t_hbm.at[:, pl.ds(bstart, _BLK)], bufs[slot], sems[slot]
        ).start()

    vec0 = idx_v[pl.ds(0, _L)]
    for j in range(_NBUF):
        fire(j, vec0[j])

    def body(v, carry):
        vec = idx_v[pl.ds(v * _L, _L)]
        vecn = idx_v[pl.ds(v * _L + _L, _L)]
        bank = lax.rem(v, 4) * _L
        @pl.when(v >= 4)
        def _():
            pltpu.make_async_copy(
                rows_v.at[pl.ds(0, _L)],
                out_hbm.at[pl.ds(base, _L)],
                sem,
            ).wait()
        for j in range(_L):
            i = v * _L + j
            slot = j % _NBUF
            pltpu.make_async_copy(
                table_t_hbm.at[:, pl.ds(0, _BLK)], bufs[slot], sems[slot]
            ).wait()
            u = vec[j]
            colv = ones * (u % _BLK)
            for g in range(_GROUPS):
                vals = plsc.load_gather(bufs[slot], [c0 + g * _L, colv])
                rows_v[bank + j, pl.ds(g * _L, _L)] = vals + rel_regs[g]
            u_next = vec[j + _NBUF] if j + _NBUF < _L else vecn[j + _NBUF - _L]
            fire(slot, u_next)
        pltpu.make_async_copy(
            rows_v.at[pl.ds(bank, _L)],
            out_hbm.at[pl.ds(base + v * _L, _L)],
            sem,
        ).start()
        return carry

    lax.fori_loop(0, _NV, body, 0)

    for j in range(_NBUF):
        pltpu.make_async_copy(
            table_t_hbm.at[:, pl.ds(0, _BLK)], bufs[j], sems[j]
        ).wait()
    for _ in range(4):
        pltpu.make_async_copy(
            rows_v.at[pl.ds(0, _L)], out_hbm.at[pl.ds(base, _L)], sem
        ).wait()


def kernel(batch_users, user_emb_weight, rel_emb_weight):
    idx = batch_users.astype(jnp.int32)
    table_t = jnp.swapaxes(user_emb_weight, 0, 1)
    rel_row = rel_emb_weight[-1]
    return _sc_lookup(table_t, idx, rel_row)
